# baseline (speedup 1.0000x reference)
import functools

import jax
import jax.numpy as jnp
from jax import lax
from jax.experimental import pallas as pl
from jax.experimental.pallas import tpu as pltpu
from jax.experimental.pallas import tpu_sc as plsc

_B = 4
_S = 2048
_D = 768
_N = _B * _S
_L = 16
_NC = 2
_NS = 16
_NW = _NC * _NS
_SBLK = _S // _NW
_PER_W = _B * _SBLK
_C = 16
_KPB = _SBLK // _C
_NCH = _PER_W // _C
_CVECS = _D // _L


def _make_sc_embed():
    mesh = plsc.VectorSubcoreMesh(core_axis_name="c", subcore_axis_name="s")

    @functools.partial(
        pl.kernel,
        mesh=mesh,
        out_type=jax.ShapeDtypeStruct((_N, _D), jnp.float32),
        scratch_types=[
            pltpu.VMEM((_NCH, _C), jnp.int32),
            pltpu.VMEM((_NCH, _C), jnp.int32),
            pltpu.VMEM((_C, _D), jnp.float32),
            pltpu.VMEM((_C, _D), jnp.float32),
            pltpu.VMEM((_SBLK, _D), jnp.float32),
            pltpu.VMEM((2, _D), jnp.float32),
            pltpu.SemaphoreType.DMA,
            pltpu.SemaphoreType.DMA,
            pltpu.SemaphoreType.DMA,
            pltpu.SemaphoreType.DMA,
            pltpu.SemaphoreType.DMA,
        ],
    )
    def sc_embed(ids_hbm, sids_hbm, tok_hbm, seg_hbm, pos_hbm, out_hbm,
                 idx_v, sidx_v, tok0, tok1, pos_v, seg_v,
                 st0, st1, spp, so0, so1):
        wid = lax.axis_index("s") * _NC + lax.axis_index("c")

        toks = (tok0, tok1)
        sts = (st0, st1)
        sos = (so0, so1)

        pos_cp = pltpu.make_async_copy(
            pos_hbm.at[pl.ds(wid * _SBLK, _SBLK)], pos_v, spp)
        pos_cp.start()
        pltpu.sync_copy(ids_hbm.at[wid], idx_v)
        pltpu.sync_copy(sids_hbm.at[wid], sidx_v)
        pltpu.sync_copy(seg_hbm, seg_v)

        def gather(j, b):
            return pltpu.make_async_copy(
                tok_hbm.at[idx_v.at[j]], toks[b], sts[b])

        def out_copy(j, b):
            row0 = ((j // _KPB) * _S + wid * _SBLK + (j % _KPB) * _C)
            return pltpu.make_async_copy(
                toks[b], out_hbm.at[pl.ds(row0, _C)], sos[b])

        gather(0, 0).start()
        pos_cp.wait()

        def outer(i, carry):
            for b in (0, 1):
                j = i * 2 + b
                nb = 1 - b
                gather(j, b).wait()

                @pl.when(j >= 1)
                def _wait_prev_out():
                    out_copy(j - 1, nb).wait()

                @pl.when(j + 1 < _NCH)
                def _issue_next():
                    gather(j + 1, nb).start()

                tok_b = toks[b]
                p0 = (j % _KPB) * _C

                svec = sidx_v[j, :].astype(jnp.float32)
                sidf = [
                    jnp.broadcast_to(svec[r], (_L,)) for r in range(_C)
                ]

                def col_body(cb, carry2):
                    sl = pl.ds(cb * _L, _L)
                    s0v = seg_v[0, sl]
                    dsv = seg_v[1, sl] - s0v
                    for r in range(_C):
                        tok_b[r, sl] = (tok_b[r, sl] + pos_v[p0 + r, sl]
                                        + (s0v + sidf[r] * dsv))
                    return carry2

                lax.fori_loop(0, _CVECS, col_body, None)
                out_copy(j, b).start()
            return carry

        lax.fori_loop(0, _NCH // 2, outer, None)
        out_copy(_NCH - 1, (_NCH - 1) % 2).wait()

    return sc_embed


_sc_embed = _make_sc_embed()


@jax.jit
def kernel(input_ids, segment_ids, token_table, segment_table,
           position_table):
    def regroup(x):
        x = x.reshape(_B, _NW, _KPB, _C).astype(jnp.int32)
        return x.transpose(1, 0, 2, 3).reshape(_NW, _NCH, _C)

    out = _sc_embed(regroup(input_ids), regroup(segment_ids),
                    token_table, segment_table, position_table)
    return out.reshape(_B, _S, _D)


# speedup vs baseline: 1.9302x; 1.9302x over previous
import functools

import jax
import jax.numpy as jnp
from jax import lax
from jax.experimental import pallas as pl
from jax.experimental.pallas import tpu as pltpu
from jax.experimental.pallas import tpu_sc as plsc

_B = 4
_S = 2048
_D = 768
_N = _B * _S
_L = 16
_NC = 2
_NS = 16
_NW = _NC * _NS
_SBLK = _S // _NW
_PER_W = _B * _SBLK
_C = 16
_KPB = _SBLK // _C
_NCH = _PER_W // _C
_CVECS = _D // _L


def _make_sc_embed():
    mesh = plsc.VectorSubcoreMesh(core_axis_name="c", subcore_axis_name="s")

    @functools.partial(
        pl.kernel,
        mesh=mesh,
        out_type=jax.ShapeDtypeStruct((_N, _D), jnp.float32),
        scratch_types=[
            pltpu.VMEM((_NCH, _C), jnp.int32),
            pltpu.VMEM((_NCH, _C), jnp.int32),
            pltpu.VMEM((_C, _D), jnp.float32),
            pltpu.VMEM((_C, _D), jnp.float32),
            pltpu.VMEM((_SBLK, _D), jnp.float32),
            pltpu.VMEM((2, _D), jnp.float32),
            pltpu.SemaphoreType.DMA,
            pltpu.SemaphoreType.DMA,
            pltpu.SemaphoreType.DMA,
            pltpu.SemaphoreType.DMA,
            pltpu.SemaphoreType.DMA,
        ],
    )
    def sc_embed(ids_hbm, sids_hbm, tok_hbm, seg_hbm, pos_hbm, out_hbm,
                 idx_v, sidx_v, tok0, tok1, pos_v, seg_v,
                 st0, st1, spp, so0, so1):
        wid = lax.axis_index("s") * _NC + lax.axis_index("c")

        toks = (tok0, tok1)
        sts = (st0, st1)
        sos = (so0, so1)

        pos_cp = pltpu.make_async_copy(
            pos_hbm.at[pl.ds(wid * _SBLK, _SBLK)], pos_v, spp)
        pos_cp.start()
        pltpu.sync_copy(ids_hbm.at[wid], idx_v)
        pltpu.sync_copy(sids_hbm.at[wid], sidx_v)
        pltpu.sync_copy(seg_hbm, seg_v)

        def gather(j, b):
            return pltpu.make_async_copy(
                tok_hbm.at[idx_v.at[j]], toks[b], sts[b])

        def out_copy(j, b):
            row0 = ((j // _KPB) * _S + wid * _SBLK + (j % _KPB) * _C)
            return pltpu.make_async_copy(
                toks[b], out_hbm.at[pl.ds(row0, _C)], sos[b])

        gather(0, 0).start()
        pos_cp.wait()

        for j in range(_NCH):
            b = j % 2
            nb = 1 - b
            gather(j, b).wait()
            if j >= 1:
                out_copy(j - 1, nb).wait()
            if j + 1 < _NCH:
                gather(j + 1, nb).start()

            tok_b = toks[b]
            p0 = (j % _KPB) * _C

            svec = sidx_v[j, :].astype(jnp.float32)
            sidf = [
                jnp.broadcast_to(svec[r], (_L,)) for r in range(_C)
            ]

            def col_body(cb, carry2, tok_b=tok_b, p0=p0, sidf=sidf):
                sl = pl.ds(cb * _L, _L)
                s0v = seg_v[0, sl]
                dsv = seg_v[1, sl] - s0v
                for r in range(_C):
                    tok_b[r, sl] = (tok_b[r, sl] + pos_v[p0 + r, sl]
                                    + (s0v + sidf[r] * dsv))
                return carry2

            lax.fori_loop(0, _CVECS, col_body, None)
            out_copy(j, b).start()

        out_copy(_NCH - 1, (_NCH - 1) % 2).wait()

    return sc_embed


_sc_embed = _make_sc_embed()


@jax.jit
def kernel(input_ids, segment_ids, token_table, segment_table,
           position_table):
    def regroup(x):
        x = x.reshape(_B, _NW, _KPB, _C).astype(jnp.int32)
        return x.transpose(1, 0, 2, 3).reshape(_NW, _NCH, _C)

    out = _sc_embed(regroup(input_ids), regroup(segment_ids),
                    token_table, segment_table, position_table)
    return out.reshape(_B, _S, _D)


# speedup vs baseline: 1.9313x; 1.0006x over previous
import functools

import jax
import jax.numpy as jnp
from jax import lax
from jax.experimental import pallas as pl
from jax.experimental.pallas import tpu as pltpu
from jax.experimental.pallas import tpu_sc as plsc

_B = 4
_S = 2048
_D = 768
_N = _B * _S
_L = 16
_NC = 2
_NS = 16
_NW = _NC * _NS
_SBLK = _S // _NW
_PER_W = _B * _SBLK
_C = 32
_KPB = _SBLK // _C
_NCH = _PER_W // _C
_CVECS = _D // _L


def _make_sc_embed():
    mesh = plsc.VectorSubcoreMesh(core_axis_name="c", subcore_axis_name="s")

    @functools.partial(
        pl.kernel,
        mesh=mesh,
        out_type=jax.ShapeDtypeStruct((_N, _D), jnp.float32),
        scratch_types=[
            pltpu.VMEM((_NCH, _C), jnp.int32),
            pltpu.VMEM((_NCH, _C), jnp.int32),
            pltpu.VMEM((_C, _D), jnp.float32),
            pltpu.VMEM((_C, _D), jnp.float32),
            pltpu.VMEM((_SBLK, _D), jnp.float32),
            pltpu.VMEM((2, _D), jnp.float32),
            pltpu.SemaphoreType.DMA,
            pltpu.SemaphoreType.DMA,
            pltpu.SemaphoreType.DMA,
            pltpu.SemaphoreType.DMA,
            pltpu.SemaphoreType.DMA,
        ],
    )
    def sc_embed(ids_hbm, sids_hbm, tok_hbm, seg_hbm, pos_hbm, out_hbm,
                 idx_v, sidx_v, tok0, tok1, pos_v, seg_v,
                 st0, st1, spp, so0, so1):
        wid = lax.axis_index("s") * _NC + lax.axis_index("c")

        toks = (tok0, tok1)
        sts = (st0, st1)
        sos = (so0, so1)

        pos_cp = pltpu.make_async_copy(
            pos_hbm.at[pl.ds(wid * _SBLK, _SBLK)], pos_v, spp)
        pos_cp.start()
        pltpu.sync_copy(ids_hbm.at[wid], idx_v)
        pltpu.sync_copy(sids_hbm.at[wid], sidx_v)
        pltpu.sync_copy(seg_hbm, seg_v)

        def gather(j, b):
            return pltpu.make_async_copy(
                tok_hbm.at[idx_v.at[j]], toks[b], sts[b])

        def out_copy(j, b):
            row0 = ((j // _KPB) * _S + wid * _SBLK + (j % _KPB) * _C)
            return pltpu.make_async_copy(
                toks[b], out_hbm.at[pl.ds(row0, _C)], sos[b])

        gather(0, 0).start()
        pos_cp.wait()

        for j in range(_NCH):
            b = j % 2
            nb = 1 - b
            gather(j, b).wait()
            if j >= 1:
                out_copy(j - 1, nb).wait()
            if j + 1 < _NCH:
                gather(j + 1, nb).start()

            tok_b = toks[b]
            p0 = (j % _KPB) * _C

            for g in range(_C // _L):
                r0 = g * _L
                svec = sidx_v[j, pl.ds(r0, _L)].astype(jnp.float32)
                sidf = [
                    jnp.broadcast_to(svec[r], (_L,)) for r in range(_L)
                ]

                def col_body(cb, carry2, tok_b=tok_b, p0=p0, r0=r0,
                             sidf=sidf):
                    sl = pl.ds(cb * _L, _L)
                    s0v = seg_v[0, sl]
                    dsv = seg_v[1, sl] - s0v
                    for r in range(_L):
                        rr = r0 + r
                        tok_b[rr, sl] = (tok_b[rr, sl] + pos_v[p0 + rr, sl]
                                         + (s0v + sidf[r] * dsv))
                    return carry2

                lax.fori_loop(0, _CVECS, col_body, None)
            out_copy(j, b).start()

        out_copy(_NCH - 1, (_NCH - 1) % 2).wait()

    return sc_embed


_sc_embed = _make_sc_embed()


@jax.jit
def kernel(input_ids, segment_ids, token_table, segment_table,
           position_table):
    def regroup(x):
        x = x.reshape(_B, _NW, _KPB, _C).astype(jnp.int32)
        return x.transpose(1, 0, 2, 3).reshape(_NW, _NCH, _C)

    out = _sc_embed(regroup(input_ids), regroup(segment_ids),
                    token_table, segment_table, position_table)
    return out.reshape(_B, _S, _D)


# speedup vs baseline: 1.9626x; 1.0162x over previous
import functools

import jax
import jax.numpy as jnp
from jax import lax
from jax.experimental import pallas as pl
from jax.experimental.pallas import tpu as pltpu
from jax.experimental.pallas import tpu_sc as plsc

_B = 4
_S = 2048
_D = 768
_N = _B * _S
_L = 16
_NC = 2
_NS = 16
_NW = _NC * _NS
_SBLK = _S // _NW
_C = 32
_KPB = _SBLK // _C
_NCH = _B * _KPB
_CVECS = _D // _L


def _make_sc_embed():
    mesh = plsc.VectorSubcoreMesh(core_axis_name="c", subcore_axis_name="s")

    @functools.partial(
        pl.kernel,
        mesh=mesh,
        out_type=jax.ShapeDtypeStruct((_N, _D), jnp.float32),
        scratch_types=[
            pltpu.VMEM((_B, _SBLK), jnp.int32),
            pltpu.VMEM((_B, _SBLK), jnp.int32),
            pltpu.VMEM((_C, _D), jnp.float32),
            pltpu.VMEM((_C, _D), jnp.float32),
            pltpu.VMEM((_SBLK, _D), jnp.float32),
            pltpu.VMEM((2, _D), jnp.float32),
            pltpu.SemaphoreType.DMA,
            pltpu.SemaphoreType.DMA,
            pltpu.SemaphoreType.DMA,
            pltpu.SemaphoreType.DMA,
            pltpu.SemaphoreType.DMA,
            pltpu.SemaphoreType.DMA,
            pltpu.SemaphoreType.DMA,
        ],
    )
    def sc_embed(ids_hbm, sids_hbm, tok_hbm, seg_hbm, pos_hbm, out_hbm,
                 idx_v, sidx_v, tok0, tok1, pos_v, seg_v,
                 st0, st1, si, ss, sgp, so0, so1):
        wid = lax.axis_index("s") * _NC + lax.axis_index("c")
        blk = wid * _SBLK

        toks = (tok0, tok1)
        sts = (st0, st1)
        sos = (so0, so1)

        id_cps = [
            pltpu.make_async_copy(
                ids_hbm.at[b, pl.ds(blk, _SBLK)], idx_v.at[b], si)
            for b in range(_B)
        ]
        sid_cps = [
            pltpu.make_async_copy(
                sids_hbm.at[b, pl.ds(blk, _SBLK)], sidx_v.at[b], ss)
            for b in range(_B)
        ]
        seg_cp = pltpu.make_async_copy(seg_hbm, seg_v, sgp)
        pos_cp = pltpu.make_async_copy(
            pos_hbm.at[pl.ds(blk, _SBLK)], pos_v, sgp)
        for cp in id_cps + sid_cps + [seg_cp, pos_cp]:
            cp.start()
        for cp in id_cps:
            cp.wait()

        def gather(j, b):
            bi, k = divmod(j, _KPB)
            return pltpu.make_async_copy(
                tok_hbm.at[idx_v.at[bi, pl.ds(k * _C, _C)]], toks[b], sts[b])

        def out_copy(j, b):
            bi, k = divmod(j, _KPB)
            row0 = bi * _S + blk + k * _C
            return pltpu.make_async_copy(
                toks[b], out_hbm.at[pl.ds(row0, _C)], sos[b])

        gather(0, 0).start()
        for cp in sid_cps:
            cp.wait()
        seg_cp.wait()
        pos_cp.wait()

        for j in range(_NCH):
            b = j % 2
            nb = 1 - b
            gather(j, b).wait()
            if j >= 1:
                out_copy(j - 1, nb).wait()
            if j + 1 < _NCH:
                gather(j + 1, nb).start()

            tok_b = toks[b]
            bi, k = divmod(j, _KPB)
            p0 = k * _C

            for g in range(_C // _L):
                r0 = g * _L
                svec = sidx_v[bi, pl.ds(p0 + r0, _L)].astype(jnp.float32)
                sidf = [
                    jnp.broadcast_to(svec[r], (_L,)) for r in range(_L)
                ]

                def col_body(cb, carry2, tok_b=tok_b, p0=p0, r0=r0,
                             sidf=sidf):
                    sl = pl.ds(cb * _L, _L)
                    s0v = seg_v[0, sl]
                    dsv = seg_v[1, sl] - s0v
                    for r in range(_L):
                        rr = r0 + r
                        tok_b[rr, sl] = (tok_b[rr, sl] + pos_v[p0 + rr, sl]
                                         + (s0v + sidf[r] * dsv))
                    return carry2

                lax.fori_loop(0, _CVECS, col_body, None)
            out_copy(j, b).start()

        out_copy(_NCH - 1, (_NCH - 1) % 2).wait()

    return sc_embed


_sc_embed = _make_sc_embed()


@jax.jit
def kernel(input_ids, segment_ids, token_table, segment_table,
           position_table):
    out = _sc_embed(input_ids.astype(jnp.int32),
                    segment_ids.astype(jnp.int32),
                    token_table, segment_table, position_table)
    return out.reshape(_B, _S, _D)


# speedup vs baseline: 2.0886x; 1.0642x over previous
import functools

import jax
import jax.numpy as jnp
from jax import lax
from jax.experimental import pallas as pl
from jax.experimental.pallas import tpu as pltpu
from jax.experimental.pallas import tpu_sc as plsc

_B = 4
_S = 2048
_D = 768
_N = _B * _S
_L = 16
_NC = 2
_NS = 16
_NW = _NC * _NS
_SBLK = _S // _NW
_C = 32
_KPB = _SBLK // _C
_NCH = _B * _KPB
_CVECS = _D // _L


def _make_sc_embed():
    mesh = plsc.VectorSubcoreMesh(core_axis_name="c", subcore_axis_name="s")

    @functools.partial(
        pl.kernel,
        mesh=mesh,
        out_type=jax.ShapeDtypeStruct((_N, _D), jnp.float32),
        scratch_types=[
            pltpu.VMEM((_B, _SBLK), jnp.int32),
            pltpu.VMEM((_B, _SBLK), jnp.int32),
            pltpu.VMEM((_C, _D), jnp.float32),
            pltpu.VMEM((_C, _D), jnp.float32),
            pltpu.VMEM((_SBLK, _D), jnp.float32),
            pltpu.VMEM((2, _D), jnp.float32),
            pltpu.SemaphoreType.DMA,
            pltpu.SemaphoreType.DMA,
            pltpu.SemaphoreType.DMA,
            pltpu.SemaphoreType.DMA,
            pltpu.SemaphoreType.DMA,
            pltpu.SemaphoreType.DMA,
            pltpu.SemaphoreType.DMA,
        ],
    )
    def sc_embed(ids_hbm, sids_hbm, tok_hbm, seg_hbm, pos_hbm, out_hbm,
                 idx_v, sidx_v, tok0, tok1, pos_v, seg_v,
                 st0, st1, si, ss, sgp, so0, so1):
        wid = lax.axis_index("s") * _NC + lax.axis_index("c")
        blk = wid * _SBLK

        toks = (tok0, tok1)
        sts = (st0, st1)
        sos = (so0, so1)

        id_cps = [
            pltpu.make_async_copy(
                ids_hbm.at[b, pl.ds(blk, _SBLK)], idx_v.at[b], si)
            for b in range(_B)
        ]
        sid_cps = [
            pltpu.make_async_copy(
                sids_hbm.at[b, pl.ds(blk, _SBLK)], sidx_v.at[b], ss)
            for b in range(_B)
        ]
        seg_cp = pltpu.make_async_copy(seg_hbm, seg_v, sgp)
        pos_cp = pltpu.make_async_copy(
            pos_hbm.at[pl.ds(blk, _SBLK)], pos_v, sgp)
        for cp in id_cps + sid_cps + [seg_cp, pos_cp]:
            cp.start()
        for cp in id_cps:
            cp.wait()

        def gather(bi, k, b):
            return pltpu.make_async_copy(
                tok_hbm.at[idx_v.at[bi, pl.ds(k * _C, _C)]], toks[b], sts[b])

        def out_copy(bi, k, b):
            row0 = bi * _S + blk + k * _C
            return pltpu.make_async_copy(
                toks[b], out_hbm.at[pl.ds(row0, _C)], sos[b])

        gather(0, 0, 0).start()
        for cp in sid_cps:
            cp.wait()
        seg_cp.wait()
        pos_cp.wait()

        def batch_body(bi, carry):
            for k in range(_KPB):
                j = bi * _KPB + k
                nb = 1 - k
                gather(bi, k, k).wait()

                @pl.when(j >= 1)
                def _wait_prev_out():
                    out_copy(bi + k - 1, 1 - k, nb).wait()

                @pl.when(j + 1 < _NCH)
                def _issue_next():
                    gather(bi + k, 1 - k, nb).start()

                tok_b = toks[k]
                p0 = k * _C

                for g in range(_C // _L):
                    r0 = g * _L
                    svec = sidx_v[bi, pl.ds(p0 + r0, _L)].astype(
                        jnp.float32)
                    sidf = [
                        jnp.broadcast_to(svec[r], (_L,)) for r in range(_L)
                    ]

                    def col_body(cb, carry2, tok_b=tok_b, p0=p0, r0=r0,
                                 sidf=sidf):
                        sl = pl.ds(cb * _L, _L)
                        s0v = seg_v[0, sl]
                        dsv = seg_v[1, sl] - s0v
                        for r in range(_L):
                            rr = r0 + r
                            tok_b[rr, sl] = (tok_b[rr, sl]
                                             + pos_v[p0 + rr, sl]
                                             + (s0v + sidf[r] * dsv))
                        return carry2

                    lax.fori_loop(0, _CVECS, col_body, None)
                out_copy(bi, k, k).start()
            return carry

        lax.fori_loop(0, _B, batch_body, None)
        out_copy(_B - 1, _KPB - 1, (_NCH - 1) % 2).wait()

    return sc_embed


_sc_embed = _make_sc_embed()


@jax.jit
def kernel(input_ids, segment_ids, token_table, segment_table,
           position_table):
    out = _sc_embed(input_ids.astype(jnp.int32),
                    segment_ids.astype(jnp.int32),
                    token_table, segment_table, position_table)
    return out.reshape(_B, _S, _D)
